# baseline (speedup 1.0000x reference)
import functools

import jax
import jax.numpy as jnp
from jax import lax
from jax.experimental import pallas as pl
from jax.experimental.pallas import tpu as pltpu
from jax.experimental.pallas import tpu_sc as plsc

N = 10000
D = 128
DP = 144
NROWS = 10112
NW = 32
CHUNK = 32
N_INC = 320000
NBUF = 6
WS = 64
NWIN = -(-((N_INC + NW * CHUNK - 1) // (NW * CHUNK)) // WS)
CHUNKS = NWIN * WS
NP = NW * CHUNKS * CHUNK
RPT = NROWS // 16


_mesh = plsc.VectorSubcoreMesh(core_axis_name="c", subcore_axis_name="s")


@functools.partial(
    pl.kernel,
    out_type=jax.ShapeDtypeStruct((2, NROWS, DP), jnp.float32),
    mesh=_mesh,
    scratch_types=[
        pltpu.VMEM_SHARED((NROWS, DP), jnp.float32),
        pltpu.VMEM((2, WS, CHUNK), jnp.int32),
        pltpu.VMEM((2, WS, CHUNK), jnp.int32),
        pltpu.VMEM((NBUF, CHUNK, DP), jnp.float32),
        pltpu.SemaphoreType.DMA((NBUF,)),
        pltpu.SemaphoreType.DMA,
    ],
    compiler_params=pltpu.CompilerParams(use_tc_tiling_on_sc=False),
)
def _sc_pass(gidx, sidx, table, zeros, out, acc, gwin, swin, bufs, gsems,
             wsem):
    cid = lax.axis_index("c")
    sid = lax.axis_index("s")
    wid = sid * 2 + cid
    pltpu.sync_copy(zeros.at[pl.ds(sid * RPT, RPT)],
                    acc.at[pl.ds(sid * RPT, RPT)])
    pltpu.sync_copy(gidx.at[wid, pl.ds(0, WS)], gwin.at[0])
    pltpu.sync_copy(sidx.at[wid, pl.ds(0, WS)], swin.at[0])
    plsc.subcore_barrier()

    def window(w, carry):
        p = lax.rem(w, 2)

        @pl.when(w + 1 < NWIN)
        def _():
            pltpu.async_copy(gidx.at[wid, pl.ds((w + 1) * WS, WS)],
                             gwin.at[1 - p], wsem)
            pltpu.async_copy(sidx.at[wid, pl.ds((w + 1) * WS, WS)],
                             swin.at[1 - p], wsem)

        for k in range(NBUF - 1):
            pltpu.async_copy(table.at[gwin.at[p, k]], bufs.at[k],
                             gsems.at[k])

        def chunk(c, carry2):
            b = lax.rem(c, NBUF)
            nc = c + NBUF - 1
            nb = lax.rem(nc, NBUF)
            pltpu.make_async_copy(
                table.at[gwin.at[p, c]], bufs.at[b], gsems.at[b]).wait()

            @pl.when(nc < WS)
            def _():
                pltpu.async_copy(table.at[gwin.at[p, nc]], bufs.at[nb],
                                 gsems.at[nb])

            pltpu.sync_copy(bufs.at[b], acc.at[swin.at[p, c]], add=True)
            return carry2

        lax.fori_loop(0, WS, chunk, 0)

        @pl.when(w + 1 < NWIN)
        def _():
            pltpu.make_async_copy(gidx.at[wid, pl.ds((w + 1) * WS, WS)],
                                  gwin.at[1 - p], wsem).wait()
            pltpu.make_async_copy(sidx.at[wid, pl.ds((w + 1) * WS, WS)],
                                  swin.at[1 - p], wsem).wait()

        return carry

    lax.fori_loop(0, NWIN, window, 0)
    plsc.subcore_barrier()
    pltpu.sync_copy(acc.at[pl.ds(sid * RPT, RPT)],
                    out.at[cid, pl.ds(sid * RPT, RPT)])


def _ones_col(rows):
    col = lax.broadcasted_iota(jnp.int32, (rows, DP - D), 1)
    return jnp.where(col == 0, 1.0, 0.0).astype(jnp.float32)


def _tc_in_body(x_ref, w_ref, o_ref):
    xw = jnp.dot(x_ref[...], w_ref[...], preferred_element_type=jnp.float32)
    top = jnp.concatenate([xw, _ones_col(N)], axis=1)
    o_ref[...] = jnp.concatenate(
        [top, jnp.zeros((NROWS - N, DP), jnp.float32)], axis=0)


def _tc_scale_body(p_ref, o_ref):
    s = p_ref[0] + p_ref[1]
    c = s[:, D:D + 1]
    binv = jnp.where(c > 0, 1.0 / c, 0.0)
    o_ref[...] = s * binv


def _tc_mid_body(p_ref, b_ref, w_ref, o_ref):
    s = p_ref[0] + p_ref[1]
    c = s[:, D:D + 1]
    dinv = jnp.where(c > 0, 1.0 / c, 0.0)
    h = jnp.maximum(s[:, :D] * dinv + b_ref[...], 0.0)
    hw = jnp.dot(h, w_ref[...], preferred_element_type=jnp.float32)
    o_ref[...] = jnp.concatenate([hw, _ones_col(NROWS)], axis=1)


def _tc_out_body(p_ref, b_ref, o_ref):
    s = p_ref[0] + p_ref[1]
    c = s[:, D:D + 1]
    dinv = jnp.where(c > 0, 1.0 / c, 0.0)
    o_ref[...] = s[:N, :D] * dinv[:N] + b_ref[...]


_tc_in = pl.pallas_call(
    _tc_in_body, out_shape=jax.ShapeDtypeStruct((NROWS, DP), jnp.float32))
_tc_scale = pl.pallas_call(
    _tc_scale_body, out_shape=jax.ShapeDtypeStruct((NROWS, DP), jnp.float32))
_tc_mid = pl.pallas_call(
    _tc_mid_body, out_shape=jax.ShapeDtypeStruct((NROWS, DP), jnp.float32))
_tc_out = pl.pallas_call(
    _tc_out_body, out_shape=jax.ShapeDtypeStruct((N, D), jnp.float32))


def kernel(x, hyperedge_index, W1, b1, W2, b2):
    src = hyperedge_index[0]
    dst = hyperedge_index[1]
    pad = jnp.full((NP - N_INC,), N, dtype=jnp.int32)
    srcp = jnp.concatenate([src, pad]).reshape(NW, CHUNKS, CHUNK)
    dstp = jnp.concatenate([dst, pad]).reshape(NW, CHUNKS, CHUNK)
    zeros = jnp.zeros((NROWS, DP), jnp.float32)

    xw1 = _tc_in(x, W1)
    macc1 = _sc_pass(srcp, dstp, xw1, zeros)
    m1 = _tc_scale(macc1)
    oacc1 = _sc_pass(dstp, srcp, m1, zeros)
    h2 = _tc_mid(oacc1, b1, W2)
    macc2 = _sc_pass(srcp, dstp, h2, zeros)
    m2 = _tc_scale(macc2)
    oacc2 = _sc_pass(dstp, srcp, m2, zeros)
    return _tc_out(oacc2, b2)


# speedup vs baseline: 2.8987x; 2.8987x over previous
import functools

import jax
import jax.numpy as jnp
from jax import lax
from jax.experimental import pallas as pl
from jax.experimental.pallas import tpu as pltpu
from jax.experimental.pallas import tpu_sc as plsc

N = 10000
D = 128
DP = 144
NROWS = 10112
NW = 32
CHUNK = 32
N_INC = 320000
NBUF = 4
CHUNKS = (N_INC + NW * CHUNK - 1) // (NW * CHUNK)
NP = NW * CHUNKS * CHUNK
RPT = NROWS // 16


_mesh = plsc.VectorSubcoreMesh(core_axis_name="c", subcore_axis_name="s")


@functools.partial(
    pl.kernel,
    out_type=jax.ShapeDtypeStruct((2, NROWS, DP), jnp.float32),
    mesh=_mesh,
    scratch_types=[
        pltpu.VMEM_SHARED((NROWS, DP), jnp.float32),
        pltpu.VMEM((CHUNKS, CHUNK), jnp.int32),
        pltpu.VMEM((CHUNKS, CHUNK), jnp.int32),
        pltpu.VMEM((NBUF, CHUNK, DP), jnp.float32),
        pltpu.SemaphoreType.DMA((NBUF,)),
    ],
    compiler_params=pltpu.CompilerParams(use_tc_tiling_on_sc=False),
)
def _sc_pass(gidx, sidx, table, zeros, out, acc, gslab, sslab, bufs, gsems):
    cid = lax.axis_index("c")
    sid = lax.axis_index("s")
    wid = sid * 2 + cid
    pltpu.sync_copy(zeros.at[pl.ds(sid * RPT, RPT)],
                    acc.at[pl.ds(sid * RPT, RPT)])
    pltpu.sync_copy(gidx.at[wid], gslab)
    pltpu.sync_copy(sidx.at[wid], sslab)
    plsc.subcore_barrier()

    for k in range(NBUF - 1):
        pltpu.async_copy(table.at[gslab.at[k]], bufs.at[k], gsems.at[k])

    def body(i, carry):
        b = lax.rem(i, NBUF)
        nxt = lax.rem(i + NBUF - 1, NBUF)
        pltpu.make_async_copy(
            table.at[gslab.at[i]], bufs.at[b], gsems.at[b]).wait()

        @pl.when(i + NBUF - 1 < CHUNKS)
        def _():
            pltpu.async_copy(table.at[gslab.at[i + NBUF - 1]],
                             bufs.at[nxt], gsems.at[nxt])

        pltpu.sync_copy(bufs.at[b], acc.at[sslab.at[i]], add=True)
        return carry

    lax.fori_loop(0, CHUNKS, body, 0)
    plsc.subcore_barrier()
    pltpu.sync_copy(acc.at[pl.ds(sid * RPT, RPT)],
                    out.at[cid, pl.ds(sid * RPT, RPT)])


def _ones_col(rows):
    col = lax.broadcasted_iota(jnp.int32, (rows, DP - D), 1)
    return jnp.where(col == 0, 1.0, 0.0).astype(jnp.float32)


def _tc_in_body(x_ref, w_ref, o_ref):
    xw = jnp.dot(x_ref[...], w_ref[...], preferred_element_type=jnp.float32)
    top = jnp.concatenate([xw, _ones_col(N)], axis=1)
    o_ref[...] = jnp.concatenate(
        [top, jnp.zeros((NROWS - N, DP), jnp.float32)], axis=0)


def _tc_scale_body(p_ref, o_ref):
    s = p_ref[0] + p_ref[1]
    c = s[:, D:D + 1]
    binv = jnp.where(c > 0, 1.0 / c, 0.0)
    o_ref[...] = s * binv


def _tc_mid_body(p_ref, b_ref, w_ref, o_ref):
    s = p_ref[0] + p_ref[1]
    c = s[:, D:D + 1]
    dinv = jnp.where(c > 0, 1.0 / c, 0.0)
    h = jnp.maximum(s[:, :D] * dinv + b_ref[...], 0.0)
    hw = jnp.dot(h, w_ref[...], preferred_element_type=jnp.float32)
    o_ref[...] = jnp.concatenate([hw, _ones_col(NROWS)], axis=1)


def _tc_out_body(p_ref, b_ref, o_ref):
    s = p_ref[0] + p_ref[1]
    c = s[:, D:D + 1]
    dinv = jnp.where(c > 0, 1.0 / c, 0.0)
    o_ref[...] = s[:N, :D] * dinv[:N] + b_ref[...]


_tc_in = pl.pallas_call(
    _tc_in_body, out_shape=jax.ShapeDtypeStruct((NROWS, DP), jnp.float32))
_tc_scale = pl.pallas_call(
    _tc_scale_body, out_shape=jax.ShapeDtypeStruct((NROWS, DP), jnp.float32))
_tc_mid = pl.pallas_call(
    _tc_mid_body, out_shape=jax.ShapeDtypeStruct((NROWS, DP), jnp.float32))
_tc_out = pl.pallas_call(
    _tc_out_body, out_shape=jax.ShapeDtypeStruct((N, D), jnp.float32))


def kernel(x, hyperedge_index, W1, b1, W2, b2):
    src = hyperedge_index[0]
    dst = hyperedge_index[1]
    pad = jnp.full((NP - N_INC,), N, dtype=jnp.int32)
    srcp = jnp.concatenate([src, pad]).reshape(NW, CHUNKS, CHUNK)
    dstp = jnp.concatenate([dst, pad]).reshape(NW, CHUNKS, CHUNK)
    zeros = jnp.zeros((NROWS, DP), jnp.float32)

    xw1 = _tc_in(x, W1)
    macc1 = _sc_pass(srcp, dstp, xw1, zeros)
    m1 = _tc_scale(macc1)
    oacc1 = _sc_pass(dstp, srcp, m1, zeros)
    h2 = _tc_mid(oacc1, b1, W2)
    macc2 = _sc_pass(srcp, dstp, h2, zeros)
    m2 = _tc_scale(macc2)
    oacc2 = _sc_pass(dstp, srcp, m2, zeros)
    return _tc_out(oacc2, b2)
